# Initial kernel scaffold; baseline (speedup 1.0000x reference)
import jax
import jax.numpy as jnp
from jax.experimental import pallas as pl


def kernel(batch_rois, kps_rcnn_cls_pred, kps_rcnn_reg_pred):
    raise NotImplementedError("write your pallas kernel here")


# speedup vs baseline: 1.9421x; 1.9421x over previous
import functools

import jax
import jax.numpy as jnp
from jax import lax
from jax.experimental import pallas as pl

_NUM_KPS = 17
_POS_DISTANCE = 4.0
_ROI_EXPAND = 1.2
_FW = 56
_FH = 56
_HW = _FW * _FH


def _decode_body(rois_ref, s_ref, d_ref, px_ref, py_ref, ms_ref):
    s = s_ref[...]
    m = jnp.max(s, axis=-1)
    iota = lax.broadcasted_iota(jnp.int32, s.shape, 2)
    idx = jnp.min(jnp.where(s == m[..., None], iota, _HW), axis=-1)
    onehot = iota == idx[..., None]
    dx = jnp.sum(jnp.where(onehot, d_ref[:, :, 0, :], 0.0), axis=-1) * _POS_DISTANCE
    dy = jnp.sum(jnp.where(onehot, d_ref[:, :, 1, :], 0.0), axis=-1) * _POS_DISTANCE
    idxf = idx.astype(jnp.float32)
    ix = idxf - jnp.floor(idxf / _FW) * _FW
    iy = jnp.floor(idxf / _FW)
    r = rois_ref[...]
    w = (r[:, 2] - r[:, 0]) * _ROI_EXPAND
    h = (r[:, 3] - r[:, 1]) * _ROI_EXPAND
    x1 = (r[:, 2] + r[:, 0]) * 0.5 - w * 0.5
    y1 = (r[:, 3] + r[:, 1]) * 0.5 - h * 0.5
    sx = _FW / (w + 1.0)
    sy = _FW / (h + 1.0)
    px_ref[...] = (ix + dx) / sx[:, None] + x1[:, None]
    py_ref[...] = (iy + dy) / sy[:, None] + y1[:, None]
    ms_ref[...] = m


@functools.partial(jax.jit, static_argnames=("block",))
def kernel(batch_rois, kps_rcnn_cls_pred, kps_rcnn_reg_pred, block=8):
    bs, r_per = batch_rois.shape[0], batch_rois.shape[1]
    n = bs * r_per
    scores = kps_rcnn_cls_pred.reshape(n, _NUM_KPS, _HW)
    deltas = kps_rcnn_reg_pred.reshape(n, _NUM_KPS, 2, _HW)
    rois = batch_rois[..., :4].reshape(n, 4)

    grid = (n // block,)
    px, py, ms = pl.pallas_call(
        _decode_body,
        grid=grid,
        in_specs=[
            pl.BlockSpec((block, 4), lambda i: (i, 0)),
            pl.BlockSpec((block, _NUM_KPS, _HW), lambda i: (i, 0, 0)),
            pl.BlockSpec((block, _NUM_KPS, 2, _HW), lambda i: (i, 0, 0, 0)),
        ],
        out_specs=[
            pl.BlockSpec((block, _NUM_KPS), lambda i: (i, 0)),
            pl.BlockSpec((block, _NUM_KPS), lambda i: (i, 0)),
            pl.BlockSpec((block, _NUM_KPS), lambda i: (i, 0)),
        ],
        out_shape=[
            jax.ShapeDtypeStruct((n, _NUM_KPS), jnp.float32),
            jax.ShapeDtypeStruct((n, _NUM_KPS), jnp.float32),
            jax.ShapeDtypeStruct((n, _NUM_KPS), jnp.float32),
        ],
    )(rois, scores, deltas)

    return jnp.stack([px, py, ms], axis=-1).reshape(bs, r_per, _NUM_KPS, 3)
